# Initial kernel scaffold; baseline (speedup 1.0000x reference)
import jax
import jax.numpy as jnp
from jax.experimental import pallas as pl


def kernel(x, edge_index, W1, b1, W2, b2):
    raise NotImplementedError("write your pallas kernel here")


# speedup vs baseline: 7.1770x; 7.1770x over previous
import functools

import jax
import jax.numpy as jnp
from jax import lax
from jax.experimental import pallas as pl
from jax.experimental.pallas import tpu as pltpu
from jax.experimental.pallas import tpu_sc as plsc

NC = 2
NS = 16
NW = NC * NS
CH = 128


def _mesh():
    return plsc.VectorSubcoreMesh(core_axis_name="c", subcore_axis_name="s")


_SC_PARAMS = pltpu.CompilerParams(use_tc_tiling_on_sc=False,
                                  needs_layout_passes=False)


def _deg_sc(dst_pad, zeros1, ones1, n_pad, ept):
    nch = ept // CH

    @functools.partial(
        pl.kernel,
        out_type=jax.ShapeDtypeStruct((NC, n_pad, 1), jnp.float32),
        mesh=_mesh(),
        compiler_params=_SC_PARAMS,
        scratch_types=[
            pltpu.VMEM((CH,), jnp.int32),
            pltpu.VMEM((CH, 1), jnp.float32),
            pltpu.VMEM_SHARED((n_pad, 1), jnp.float32),
        ],
    )
    def deg_k(dst_hbm, z_hbm, ones_hbm, out_hbm, idx_v, ones_v, acc):
        cid = lax.axis_index("c")
        sid = lax.axis_index("s")
        wid = cid * NS + sid
        rpt = n_pad // NS

        pltpu.sync_copy(z_hbm.at[pl.ds(sid * rpt, rpt), :],
                        acc.at[pl.ds(sid * rpt, rpt), :])
        pltpu.sync_copy(ones_hbm, ones_v)
        plsc.subcore_barrier()

        def body(i, carry):
            base = wid * ept + i * CH
            pltpu.sync_copy(dst_hbm.at[pl.ds(base, CH)], idx_v)
            pltpu.sync_copy(ones_v, acc.at[idx_v], add=True)
            return carry

        lax.fori_loop(0, nch, body, 0)
        plsc.subcore_barrier()
        pltpu.sync_copy(acc.at[pl.ds(sid * rpt, rpt), :],
                        out_hbm.at[cid, pl.ds(sid * rpt, rpt), :])

    return deg_k(dst_pad, zeros1, ones1)


def _scatter_sc(u, src_pad, dst_pad, zeros, n_pad, ept, d):
    nch = ept // CH

    @functools.partial(
        pl.kernel,
        out_type=jax.ShapeDtypeStruct((NC, n_pad, d), jnp.float32),
        mesh=_mesh(),
        compiler_params=_SC_PARAMS,
        scratch_types=[
            pltpu.VMEM((CH,), jnp.int32),
            pltpu.VMEM((CH,), jnp.int32),
            pltpu.VMEM((CH, d), jnp.float32),
            pltpu.SemaphoreType.DMA,
            pltpu.VMEM_SHARED((n_pad, d), jnp.float32),
        ],
    )
    def scat_k(u_hbm, src_hbm, dst_hbm, z_hbm, out_hbm, si_v, di_v, rows_v,
               sem, acc):
        cid = lax.axis_index("c")
        sid = lax.axis_index("s")
        wid = cid * NS + sid
        rpt = n_pad // NS

        pltpu.sync_copy(z_hbm.at[pl.ds(sid * rpt, rpt), :],
                        acc.at[pl.ds(sid * rpt, rpt), :])
        plsc.subcore_barrier()

        def body(i, carry):
            base = wid * ept + i * CH
            pltpu.sync_copy(src_hbm.at[pl.ds(base, CH)], si_v)
            pltpu.sync_copy(dst_hbm.at[pl.ds(base, CH)], di_v)
            pltpu.async_copy(u_hbm.at[si_v], rows_v, sem).wait()
            pltpu.sync_copy(rows_v, acc.at[di_v], add=True)
            return carry

        lax.fori_loop(0, nch, body, 0)
        plsc.subcore_barrier()
        pltpu.sync_copy(acc.at[pl.ds(sid * rpt, rpt), :],
                        out_hbm.at[cid, pl.ds(sid * rpt, rpt), :])

    return scat_k(u, src_pad, dst_pad, zeros)


def _decode_sc(z, src_pad, dst_pad, e_pad, ept, o):
    nch = ept // CH

    @functools.partial(
        pl.kernel,
        out_type=jax.ShapeDtypeStruct((e_pad,), jnp.float32),
        mesh=_mesh(),
        compiler_params=_SC_PARAMS,
        scratch_types=[
            pltpu.VMEM((CH,), jnp.int32),
            pltpu.VMEM((CH,), jnp.int32),
            pltpu.VMEM((CH, o), jnp.float32),
            pltpu.VMEM((CH, o), jnp.float32),
            pltpu.VMEM((CH,), jnp.float32),
            pltpu.SemaphoreType.DMA,
            pltpu.SemaphoreType.DMA,
        ],
    )
    def dec_k(z_hbm, src_hbm, dst_hbm, out_hbm, si_v, di_v, zi_v, zj_v, ob_v,
              sem_i, sem_j):
        cid = lax.axis_index("c")
        sid = lax.axis_index("s")
        wid = cid * NS + sid
        iota = lax.iota(jnp.int32, 16)

        def body(i, carry):
            base = wid * ept + i * CH
            pltpu.sync_copy(src_hbm.at[pl.ds(base, CH)], si_v)
            pltpu.sync_copy(dst_hbm.at[pl.ds(base, CH)], di_v)
            ci = pltpu.async_copy(z_hbm.at[si_v], zi_v, sem_i)
            cj = pltpu.async_copy(z_hbm.at[di_v], zj_v, sem_j)
            ci.wait()
            cj.wait()
            for g in range(CH // 16):
                rows = iota + (g * 16)
                acc = jnp.zeros((16,), jnp.float32)
                for c in range(o):
                    col = jnp.full((16,), c, jnp.int32)
                    acc = acc + (plsc.load_gather(zi_v, [rows, col]) *
                                 plsc.load_gather(zj_v, [rows, col]))
                ob_v[pl.ds(g * 16, 16)] = acc
            pltpu.sync_copy(ob_v, out_hbm.at[pl.ds(base, CH)])
            return carry

        lax.fori_loop(0, nch, body, 0)

    return dec_k(z, src_pad, dst_pad)


def _encode_tc(deg_parts, x_p, W1, n_pad):
    bn = 1024
    f, h = W1.shape

    def body(deg_ref, x_ref, w_ref, u1_ref, dinv_ref):
        deg = deg_ref[0, :, 0] + deg_ref[1, :, 0] + 1.0
        dinv = lax.rsqrt(deg)[:, None]
        h0 = jnp.dot(x_ref[...], w_ref[...], preferred_element_type=jnp.float32)
        u1_ref[...] = dinv * h0
        dinv_ref[...] = dinv

    return pl.pallas_call(
        body,
        grid=(n_pad // bn,),
        in_specs=[
            pl.BlockSpec((NC, bn, 1), lambda i: (0, i, 0)),
            pl.BlockSpec((bn, f), lambda i: (i, 0)),
            pl.BlockSpec((f, h), lambda i: (0, 0)),
        ],
        out_specs=[
            pl.BlockSpec((bn, h), lambda i: (i, 0)),
            pl.BlockSpec((bn, 1), lambda i: (i, 0)),
        ],
        out_shape=[
            jax.ShapeDtypeStruct((n_pad, h), jnp.float32),
            jax.ShapeDtypeStruct((n_pad, 1), jnp.float32),
        ],
    )(deg_parts, x_p, W1)


def _mid_tc(p, u1, dinv, b1r, W2, n_pad):
    bn = 1024
    h, o = W2.shape

    def body(p_ref, u1_ref, dinv_ref, b1_ref, w2_ref, u2_ref):
        dv = dinv_ref[...]
        hpre = dv * (p_ref[0] + p_ref[1] + u1_ref[...]) + b1_ref[...]
        hh = jnp.maximum(hpre, 0.0)
        u2_ref[...] = dv * jnp.dot(hh, w2_ref[...],
                                   preferred_element_type=jnp.float32)

    return pl.pallas_call(
        body,
        grid=(n_pad // bn,),
        in_specs=[
            pl.BlockSpec((NC, bn, h), lambda i: (0, i, 0)),
            pl.BlockSpec((bn, h), lambda i: (i, 0)),
            pl.BlockSpec((bn, 1), lambda i: (i, 0)),
            pl.BlockSpec((1, h), lambda i: (0, 0)),
            pl.BlockSpec((h, o), lambda i: (0, 0)),
        ],
        out_specs=pl.BlockSpec((bn, o), lambda i: (i, 0)),
        out_shape=jax.ShapeDtypeStruct((n_pad, o), jnp.float32),
    )(p, u1, dinv, b1r, W2)


def _final_tc(q, u2, dinv, b2r, n_pad):
    bn = 1024
    o = u2.shape[1]

    def body(q_ref, u2_ref, dinv_ref, b2_ref, z_ref):
        z_ref[...] = (dinv_ref[...] * (q_ref[0] + q_ref[1] + u2_ref[...])
                      + b2_ref[...])

    return pl.pallas_call(
        body,
        grid=(n_pad // bn,),
        in_specs=[
            pl.BlockSpec((NC, bn, o), lambda i: (0, i, 0)),
            pl.BlockSpec((bn, o), lambda i: (i, 0)),
            pl.BlockSpec((bn, 1), lambda i: (i, 0)),
            pl.BlockSpec((1, o), lambda i: (0, 0)),
        ],
        out_specs=pl.BlockSpec((bn, o), lambda i: (i, 0)),
        out_shape=jax.ShapeDtypeStruct((n_pad, o), jnp.float32),
    )(q, u2, dinv, b2r)


def kernel(x, edge_index, W1, b1, W2, b2):
    n, f = x.shape
    e = edge_index.shape[1]
    h = W1.shape[1]
    o = W2.shape[1]

    ept = -(-e // (NW * CH)) * CH
    e_pad = ept * NW
    n_pad = -(-(n + 1) // 1024) * 1024

    x_p = jnp.pad(x, ((0, n_pad - n), (0, 0)))
    src = jnp.pad(edge_index[0], (0, e_pad - e))
    dst = jnp.pad(edge_index[1], (0, e_pad - e), constant_values=n)

    zeros1 = jnp.zeros((n_pad, 1), jnp.float32)
    ones1 = jnp.ones((CH, 1), jnp.float32)
    zeros_h = jnp.zeros((n_pad, h), jnp.float32)
    zeros_o = jnp.zeros((n_pad, o), jnp.float32)

    deg_parts = _deg_sc(dst, zeros1, ones1, n_pad, ept)
    u1, dinv = _encode_tc(deg_parts, x_p, W1, n_pad)
    p = _scatter_sc(u1, src, dst, zeros_h, n_pad, ept, h)
    u2 = _mid_tc(p, u1, dinv, b1.reshape(1, -1), W2, n_pad)
    q = _scatter_sc(u2, src, dst, zeros_o, n_pad, ept, o)
    z = _final_tc(q, u2, dinv, b2.reshape(1, -1), n_pad)
    value = _decode_sc(z, src, dst, e_pad, ept, o)
    return value[:e]


# speedup vs baseline: 8.3660x; 1.1657x over previous
import functools

import jax
import jax.numpy as jnp
from jax import lax
from jax.experimental import pallas as pl
from jax.experimental.pallas import tpu as pltpu
from jax.experimental.pallas import tpu_sc as plsc

NC = 2
NS = 16
NW = NC * NS
CH = 128
NBUF = 4


def _mesh():
    return plsc.VectorSubcoreMesh(core_axis_name="c", subcore_axis_name="s")


_SC_PARAMS = pltpu.CompilerParams(use_tc_tiling_on_sc=False,
                                  needs_layout_passes=False)


def _deg_sc(dst_flat, n_pad, nch):
    ept = nch * CH

    @functools.partial(
        pl.kernel,
        out_type=jax.ShapeDtypeStruct((NW, n_pad), jnp.float32),
        mesh=_mesh(),
        compiler_params=_SC_PARAMS,
        scratch_types=[
            pltpu.VMEM((CH,), jnp.int32),
            pltpu.VMEM((n_pad,), jnp.float32),
        ],
    )
    def deg_k(dst_hbm, out_hbm, idx_v, local):
        cid = lax.axis_index("c")
        sid = lax.axis_index("s")
        wid = cid * NS + sid
        zero16 = jnp.zeros((16,), jnp.float32)
        one16 = jnp.ones((16,), jnp.float32)

        def zbody(i, carry):
            local[pl.ds(pl.multiple_of(i * 16, 16), 16)] = zero16
            return carry

        lax.fori_loop(0, n_pad // 16, zbody, 0)

        def body(i, carry):
            base = wid * ept + i * CH
            pltpu.sync_copy(dst_hbm.at[pl.ds(base, CH)], idx_v)
            for j in range(CH // 16):
                v = idx_v[pl.ds(j * 16, 16)]
                plsc.addupdate_scatter(local, [v], one16)
            return carry

        lax.fori_loop(0, nch, body, 0)
        pltpu.sync_copy(local, out_hbm.at[wid])

    return deg_k(dst_flat)


def _scatter_sc(u, src_pad, dst_pad, zeros, n_pad, nch, d):
    nb = 2

    @functools.partial(
        pl.kernel,
        out_type=jax.ShapeDtypeStruct((NC, n_pad, d), jnp.float32),
        mesh=_mesh(),
        compiler_params=_SC_PARAMS,
        scratch_types=[
            pltpu.VMEM((nch, CH), jnp.int32),
            pltpu.VMEM((nb, CH), jnp.int32),
            pltpu.VMEM((nb, CH, d), jnp.float32),
            pltpu.SemaphoreType.DMA((nb,)),
            pltpu.SemaphoreType.DMA((nb,)),
            pltpu.SemaphoreType.DMA((nb,)),
            pltpu.VMEM_SHARED((n_pad, d), jnp.float32),
        ],
    )
    def scat_k(u_hbm, src_hbm, dst_hbm, z_hbm, out_hbm, sidx, didx, rows,
               gsem, dsem, ssem, acc):
        cid = lax.axis_index("c")
        sid = lax.axis_index("s")
        wid = cid * NS + sid
        rpt = n_pad // NS

        pltpu.sync_copy(z_hbm.at[pl.ds(sid * rpt, rpt), :],
                        acc.at[pl.ds(sid * rpt, rpt), :])
        pltpu.sync_copy(src_hbm.at[wid], sidx)
        plsc.subcore_barrier()

        def g_issue(ci, b):
            pltpu.async_copy(dst_hbm.at[wid, ci], didx.at[b], dsem.at[b])
            pltpu.async_copy(u_hbm.at[sidx.at[ci]], rows.at[b], gsem.at[b])

        def g_wait(ci, b):
            pltpu.make_async_copy(dst_hbm.at[wid, ci], didx.at[b],
                                  dsem.at[b]).wait()
            pltpu.make_async_copy(u_hbm.at[sidx.at[ci]], rows.at[b],
                                  gsem.at[b]).wait()

        def s_issue(b):
            pltpu.async_copy(rows.at[b], acc.at[didx.at[b]], ssem.at[b],
                             add=True)

        def s_wait(b):
            pltpu.make_async_copy(rows.at[b], acc.at[didx.at[b]],
                                  ssem.at[b]).wait()

        g_issue(0, 0)
        g_issue(1, 1)
        g_wait(0, 0)
        s_issue(0)

        def body(k, carry):
            for u_ in range(2):
                i = 1 + k * 2 + u_
                p = (1 + u_) % 2
                q = u_ % 2
                s_wait(q)
                g_issue(i + 1, q)
                g_wait(i, p)
                s_issue(p)
            return carry

        lax.fori_loop(0, (nch - 2) // 2, body, 0)

        b = (nch - 1) % 2
        g_wait(nch - 1, b)
        s_issue(b)
        s_wait(nch % 2)
        s_wait(b)

        plsc.subcore_barrier()
        pltpu.sync_copy(acc.at[pl.ds(sid * rpt, rpt), :],
                        out_hbm.at[cid, pl.ds(sid * rpt, rpt), :])

    return scat_k(u, src_pad, dst_pad, zeros)


def _decode_sc(z, src_pad, dst_pad, nch, o):
    @functools.partial(
        pl.kernel,
        out_type=jax.ShapeDtypeStruct((NW, nch, CH), jnp.float32),
        mesh=_mesh(),
        compiler_params=_SC_PARAMS,
        scratch_types=[
            pltpu.VMEM((nch, CH), jnp.int32),
            pltpu.VMEM((nch, CH), jnp.int32),
            pltpu.VMEM((NBUF, CH, o), jnp.float32),
            pltpu.VMEM((NBUF, CH, o), jnp.float32),
            pltpu.VMEM((NBUF, CH), jnp.float32),
            pltpu.SemaphoreType.DMA((NBUF,)),
            pltpu.SemaphoreType.DMA((NBUF,)),
            pltpu.SemaphoreType.DMA((NBUF,)),
        ],
    )
    def dec_k(z_hbm, src_hbm, dst_hbm, out_hbm, sidx, didx, zi, zj, ob,
              gsi, gsj, osem):
        cid = lax.axis_index("c")
        sid = lax.axis_index("s")
        wid = cid * NS + sid
        iota = lax.iota(jnp.int32, 16)

        pltpu.sync_copy(src_hbm.at[wid], sidx)
        pltpu.sync_copy(dst_hbm.at[wid], didx)

        def g_issue(ci, b):
            pltpu.async_copy(z_hbm.at[sidx.at[ci]], zi.at[b], gsi.at[b])
            pltpu.async_copy(z_hbm.at[didx.at[ci]], zj.at[b], gsj.at[b])

        def g_wait(ci, b):
            pltpu.make_async_copy(z_hbm.at[sidx.at[ci]], zi.at[b],
                                  gsi.at[b]).wait()
            pltpu.make_async_copy(z_hbm.at[didx.at[ci]], zj.at[b],
                                  gsj.at[b]).wait()

        def compute(ci, b):
            zi_b, zj_b, ob_b = zi.at[b], zj.at[b], ob.at[b]

            def gbody(g, carry):
                rows16 = iota + g * 16
                acc = jnp.zeros((16,), jnp.float32)
                for c in range(o):
                    col = jnp.full((16,), c, jnp.int32)
                    acc = acc + (plsc.load_gather(zi_b, [rows16, col]) *
                                 plsc.load_gather(zj_b, [rows16, col]))
                ob_b[pl.ds(pl.multiple_of(g * 16, 16), 16)] = acc
                return carry

            lax.fori_loop(0, CH // 16, gbody, 0)
            pltpu.async_copy(ob_b, out_hbm.at[wid, ci], osem.at[b])

        def o_wait(b):
            pltpu.make_async_copy(ob.at[b], out_hbm.at[wid, 0],
                                  osem.at[b]).wait()

        g_issue(0, 0)
        g_issue(1, 1)
        for i in range(4):
            g_issue(i + 2, (i + 2) % 4)
            g_wait(i, i)
            compute(i, i)

        def body(k, carry):
            for u_ in range(4):
                i = 4 + k * 4 + u_
                o_wait(u_)
                g_issue(i + 2, (u_ + 2) % 4)
                g_wait(i, u_)
                compute(i, u_)
            return carry

        lax.fori_loop(0, (nch - 8) // 4, body, 0)

        for i in range(nch - 4, nch):
            b = i % 4
            o_wait(b)
            if i + 2 < nch:
                g_issue(i + 2, (i + 2) % 4)
            g_wait(i, b)
            compute(i, b)
        for b in range(4):
            o_wait(b)

    return dec_k(z, src_pad, dst_pad)


def _encode_tc(deg_parts, x_p, W1, n_pad):
    bn = 1024
    f, h = W1.shape

    def body(deg_ref, x_ref, w_ref, u1_ref, dinv_ref):
        deg = jnp.sum(deg_ref[...], axis=0) + 1.0
        dinv = lax.rsqrt(deg)[:, None]
        h0 = jnp.dot(x_ref[...], w_ref[...], preferred_element_type=jnp.float32)
        u1_ref[...] = dinv * h0
        dinv_ref[...] = dinv

    return pl.pallas_call(
        body,
        grid=(n_pad // bn,),
        in_specs=[
            pl.BlockSpec((NW, bn), lambda i: (0, i)),
            pl.BlockSpec((bn, f), lambda i: (i, 0)),
            pl.BlockSpec((f, h), lambda i: (0, 0)),
        ],
        out_specs=[
            pl.BlockSpec((bn, h), lambda i: (i, 0)),
            pl.BlockSpec((bn, 1), lambda i: (i, 0)),
        ],
        out_shape=[
            jax.ShapeDtypeStruct((n_pad, h), jnp.float32),
            jax.ShapeDtypeStruct((n_pad, 1), jnp.float32),
        ],
    )(deg_parts, x_p, W1)


def _mid_tc(p, u1, dinv, b1r, W2, n_pad):
    bn = 1024
    h, o = W2.shape

    def body(p_ref, u1_ref, dinv_ref, b1_ref, w2_ref, u2_ref):
        dv = dinv_ref[...]
        hpre = dv * (p_ref[0] + p_ref[1] + u1_ref[...]) + b1_ref[...]
        hh = jnp.maximum(hpre, 0.0)
        u2_ref[...] = dv * jnp.dot(hh, w2_ref[...],
                                   preferred_element_type=jnp.float32)

    return pl.pallas_call(
        body,
        grid=(n_pad // bn,),
        in_specs=[
            pl.BlockSpec((NC, bn, h), lambda i: (0, i, 0)),
            pl.BlockSpec((bn, h), lambda i: (i, 0)),
            pl.BlockSpec((bn, 1), lambda i: (i, 0)),
            pl.BlockSpec((1, h), lambda i: (0, 0)),
            pl.BlockSpec((h, o), lambda i: (0, 0)),
        ],
        out_specs=pl.BlockSpec((bn, o), lambda i: (i, 0)),
        out_shape=jax.ShapeDtypeStruct((n_pad, o), jnp.float32),
    )(p, u1, dinv, b1r, W2)


def _final_tc(q, u2, dinv, b2r, n_pad):
    bn = 1024
    o = u2.shape[1]

    def body(q_ref, u2_ref, dinv_ref, b2_ref, z_ref):
        z_ref[...] = (dinv_ref[...] * (q_ref[0] + q_ref[1] + u2_ref[...])
                      + b2_ref[...])

    return pl.pallas_call(
        body,
        grid=(n_pad // bn,),
        in_specs=[
            pl.BlockSpec((NC, bn, o), lambda i: (0, i, 0)),
            pl.BlockSpec((bn, o), lambda i: (i, 0)),
            pl.BlockSpec((bn, 1), lambda i: (i, 0)),
            pl.BlockSpec((1, o), lambda i: (0, 0)),
        ],
        out_specs=pl.BlockSpec((bn, o), lambda i: (i, 0)),
        out_shape=jax.ShapeDtypeStruct((n_pad, o), jnp.float32),
    )(q, u2, dinv, b2r)


def kernel(x, edge_index, W1, b1, W2, b2):
    n, f = x.shape
    e = edge_index.shape[1]
    h = W1.shape[1]
    o = W2.shape[1]

    grain = NW * CH * NBUF
    e_pad = -(-e // grain) * grain
    nch = e_pad // (NW * CH)
    n_pad = -(-(n + 1) // 1024) * 1024

    x_p = jnp.pad(x, ((0, n_pad - n), (0, 0)))
    src = jnp.pad(edge_index[0], (0, e_pad - e)).reshape(NW, nch, CH)
    dst = jnp.pad(edge_index[1], (0, e_pad - e),
                  constant_values=n).reshape(NW, nch, CH)

    zeros_h = jnp.zeros((n_pad, h), jnp.float32)
    zeros_o = jnp.zeros((n_pad, o), jnp.float32)

    deg_parts = _deg_sc(dst.reshape(-1), n_pad, nch)
    u1, dinv = _encode_tc(deg_parts, x_p, W1, n_pad)
    p = _scatter_sc(u1, src, dst, zeros_h, n_pad, nch, h)
    u2 = _mid_tc(p, u1, dinv, b1.reshape(1, -1), W2, n_pad)
    q = _scatter_sc(u2, src, dst, zeros_o, n_pad, nch, o)
    z = _final_tc(q, u2, dinv, b2.reshape(1, -1), n_pad)
    value = _decode_sc(z, src, dst, nch, o)
    return value.reshape(-1)[:e]


# speedup vs baseline: 10.9909x; 1.3138x over previous
import functools

import jax
import jax.numpy as jnp
from jax import lax
from jax.experimental import pallas as pl
from jax.experimental.pallas import tpu as pltpu
from jax.experimental.pallas import tpu_sc as plsc

NC = 2
NS = 16
NW = NC * NS
CH = 128
NBUF = 4


def _mesh():
    return plsc.VectorSubcoreMesh(core_axis_name="c", subcore_axis_name="s")


_SC_PARAMS = pltpu.CompilerParams(use_tc_tiling_on_sc=False,
                                  needs_layout_passes=False)


def _deg_sc(dst_flat, n_pad, nch):
    ept = nch * CH

    @functools.partial(
        pl.kernel,
        out_type=jax.ShapeDtypeStruct((NW, n_pad), jnp.float32),
        mesh=_mesh(),
        compiler_params=_SC_PARAMS,
        scratch_types=[
            pltpu.VMEM((CH,), jnp.int32),
            pltpu.VMEM((n_pad,), jnp.float32),
        ],
    )
    def deg_k(dst_hbm, out_hbm, idx_v, local):
        cid = lax.axis_index("c")
        sid = lax.axis_index("s")
        wid = cid * NS + sid
        zero16 = jnp.zeros((16,), jnp.float32)
        one16 = jnp.ones((16,), jnp.float32)

        def zbody(i, carry):
            local[pl.ds(pl.multiple_of(i * 16, 16), 16)] = zero16
            return carry

        lax.fori_loop(0, n_pad // 16, zbody, 0)

        def body(i, carry):
            base = wid * ept + i * CH
            pltpu.sync_copy(dst_hbm.at[pl.ds(base, CH)], idx_v)
            for j in range(CH // 16):
                v = idx_v[pl.ds(j * 16, 16)]
                plsc.addupdate_scatter(local, [v], one16)
            return carry

        lax.fori_loop(0, nch, body, 0)
        pltpu.sync_copy(local, out_hbm.at[wid])

    return deg_k(dst_flat)


def _scatter_sc(u, src_pad, dst_pad, zeros, n_pad, nch, d):
    nb = 2

    @functools.partial(
        pl.kernel,
        out_type=jax.ShapeDtypeStruct((NC, n_pad, d), jnp.float32),
        mesh=_mesh(),
        compiler_params=_SC_PARAMS,
        scratch_types=[
            pltpu.VMEM((nch, CH), jnp.int32),
            pltpu.VMEM((nb, CH), jnp.int32),
            pltpu.VMEM((nb, CH, d), jnp.float32),
            pltpu.SemaphoreType.DMA((nb,)),
            pltpu.SemaphoreType.DMA((nb,)),
            pltpu.SemaphoreType.DMA((nb,)),
            pltpu.VMEM_SHARED((n_pad, d), jnp.float32),
        ],
    )
    def scat_k(u_hbm, src_hbm, dst_hbm, z_hbm, out_hbm, sidx, didx, rows,
               gsem, dsem, ssem, acc):
        cid = lax.axis_index("c")
        sid = lax.axis_index("s")
        wid = cid * NS + sid
        rpt = n_pad // NS

        pltpu.sync_copy(z_hbm.at[pl.ds(sid * rpt, rpt), :],
                        acc.at[pl.ds(sid * rpt, rpt), :])
        pltpu.sync_copy(src_hbm.at[wid], sidx)
        plsc.subcore_barrier()

        def g_issue(ci, b):
            pltpu.async_copy(dst_hbm.at[wid, ci], didx.at[b], dsem.at[b])
            pltpu.async_copy(u_hbm.at[sidx.at[ci]], rows.at[b], gsem.at[b])

        def g_wait(ci, b):
            pltpu.make_async_copy(dst_hbm.at[wid, ci], didx.at[b],
                                  dsem.at[b]).wait()
            pltpu.make_async_copy(u_hbm.at[sidx.at[ci]], rows.at[b],
                                  gsem.at[b]).wait()

        def s_issue(b):
            pltpu.async_copy(rows.at[b], acc.at[didx.at[b]], ssem.at[b],
                             add=True)

        def s_wait(b):
            pltpu.make_async_copy(rows.at[b], acc.at[didx.at[b]],
                                  ssem.at[b]).wait()

        g_issue(0, 0)
        g_issue(1, 1)
        g_wait(0, 0)
        s_issue(0)

        def body(k, carry):
            for u_ in range(2):
                i = 1 + k * 2 + u_
                p = (1 + u_) % 2
                q = u_ % 2
                s_wait(q)
                g_issue(i + 1, q)
                g_wait(i, p)
                s_issue(p)
            return carry

        lax.fori_loop(0, (nch - 2) // 2, body, 0)

        b = (nch - 1) % 2
        g_wait(nch - 1, b)
        s_issue(b)
        s_wait(nch % 2)
        s_wait(b)

        plsc.subcore_barrier()
        pltpu.sync_copy(acc.at[pl.ds(sid * rpt, rpt), :],
                        out_hbm.at[cid, pl.ds(sid * rpt, rpt), :])

    return scat_k(u, src_pad, dst_pad, zeros)


def _decode_sc(z, src_pad, dst_pad, nch, o):
    @functools.partial(
        pl.kernel,
        out_type=jax.ShapeDtypeStruct((NW, nch, CH), jnp.float32),
        mesh=_mesh(),
        compiler_params=_SC_PARAMS,
        scratch_types=[
            pltpu.VMEM((nch, CH), jnp.int32),
            pltpu.VMEM((nch, CH), jnp.int32),
            pltpu.VMEM((NBUF, CH, o), jnp.float32),
            pltpu.VMEM((NBUF, CH, o), jnp.float32),
            pltpu.VMEM((NBUF, CH), jnp.float32),
            pltpu.SemaphoreType.DMA((NBUF,)),
            pltpu.SemaphoreType.DMA((NBUF,)),
            pltpu.SemaphoreType.DMA((NBUF,)),
        ],
    )
    def dec_k(z_hbm, src_hbm, dst_hbm, out_hbm, sidx, didx, zi, zj, ob,
              gsi, gsj, osem):
        cid = lax.axis_index("c")
        sid = lax.axis_index("s")
        wid = cid * NS + sid
        iota = lax.iota(jnp.int32, 16)

        pltpu.sync_copy(src_hbm.at[wid], sidx)
        pltpu.sync_copy(dst_hbm.at[wid], didx)

        def g_issue(ci, b):
            pltpu.async_copy(z_hbm.at[sidx.at[ci]], zi.at[b], gsi.at[b])
            pltpu.async_copy(z_hbm.at[didx.at[ci]], zj.at[b], gsj.at[b])

        def g_wait(ci, b):
            pltpu.make_async_copy(z_hbm.at[sidx.at[ci]], zi.at[b],
                                  gsi.at[b]).wait()
            pltpu.make_async_copy(z_hbm.at[didx.at[ci]], zj.at[b],
                                  gsj.at[b]).wait()

        def compute(ci, b):
            zi_b, zj_b, ob_b = zi.at[b], zj.at[b], ob.at[b]

            def gbody(g, carry):
                rows16 = iota + g * 16

                def cbody(cb, acc):
                    cbase = iota + cb * 8
                    for j in range(8):
                        col = (cbase + j) & (o - 1)
                        acc = acc + (plsc.load_gather(zi_b, [rows16, col]) *
                                     plsc.load_gather(zj_b, [rows16, col]))
                    return acc

                acc = lax.fori_loop(0, o // 8, cbody,
                                    jnp.zeros((16,), jnp.float32))
                ob_b[pl.ds(pl.multiple_of(g * 16, 16), 16)] = acc
                return carry

            lax.fori_loop(0, CH // 16, gbody, 0)
            pltpu.async_copy(ob_b, out_hbm.at[wid, ci], osem.at[b])

        def o_wait(b):
            pltpu.make_async_copy(ob.at[b], out_hbm.at[wid, 0],
                                  osem.at[b]).wait()

        g_issue(0, 0)
        g_issue(1, 1)
        for i in range(4):
            g_issue(i + 2, (i + 2) % 4)
            g_wait(i, i)
            compute(i, i)

        def body(k, carry):
            for u_ in range(4):
                i = 4 + k * 4 + u_
                o_wait(u_)
                g_issue(i + 2, (u_ + 2) % 4)
                g_wait(i, u_)
                compute(i, u_)
            return carry

        lax.fori_loop(0, (nch - 8) // 4, body, 0)

        for i in range(nch - 4, nch):
            b = i % 4
            o_wait(b)
            if i + 2 < nch:
                g_issue(i + 2, (i + 2) % 4)
            g_wait(i, b)
            compute(i, b)
        for b in range(4):
            o_wait(b)

    return dec_k(z, src_pad, dst_pad)


def _encode_tc(deg_parts, x_p, W1, n_pad):
    bn = 1024
    f, h = W1.shape

    def body(deg_ref, x_ref, w_ref, u1_ref, dinv_ref):
        deg = jnp.sum(deg_ref[...], axis=0) + 1.0
        dinv = lax.rsqrt(deg)[:, None]
        h0 = jnp.dot(x_ref[...], w_ref[...], preferred_element_type=jnp.float32)
        u1_ref[...] = dinv * h0
        dinv_ref[...] = dinv

    return pl.pallas_call(
        body,
        grid=(n_pad // bn,),
        in_specs=[
            pl.BlockSpec((NW, bn), lambda i: (0, i)),
            pl.BlockSpec((bn, f), lambda i: (i, 0)),
            pl.BlockSpec((f, h), lambda i: (0, 0)),
        ],
        out_specs=[
            pl.BlockSpec((bn, h), lambda i: (i, 0)),
            pl.BlockSpec((bn, 1), lambda i: (i, 0)),
        ],
        out_shape=[
            jax.ShapeDtypeStruct((n_pad, h), jnp.float32),
            jax.ShapeDtypeStruct((n_pad, 1), jnp.float32),
        ],
    )(deg_parts, x_p, W1)


def _mid_tc(p, u1, dinv, b1r, W2, n_pad):
    bn = 1024
    h, o = W2.shape

    def body(p_ref, u1_ref, dinv_ref, b1_ref, w2_ref, u2_ref):
        dv = dinv_ref[...]
        hpre = dv * (p_ref[0] + p_ref[1] + u1_ref[...]) + b1_ref[...]
        hh = jnp.maximum(hpre, 0.0)
        u2_ref[...] = dv * jnp.dot(hh, w2_ref[...],
                                   preferred_element_type=jnp.float32)

    return pl.pallas_call(
        body,
        grid=(n_pad // bn,),
        in_specs=[
            pl.BlockSpec((NC, bn, h), lambda i: (0, i, 0)),
            pl.BlockSpec((bn, h), lambda i: (i, 0)),
            pl.BlockSpec((bn, 1), lambda i: (i, 0)),
            pl.BlockSpec((1, h), lambda i: (0, 0)),
            pl.BlockSpec((h, o), lambda i: (0, 0)),
        ],
        out_specs=pl.BlockSpec((bn, o), lambda i: (i, 0)),
        out_shape=jax.ShapeDtypeStruct((n_pad, o), jnp.float32),
    )(p, u1, dinv, b1r, W2)


def _final_tc(q, u2, dinv, b2r, n_pad):
    bn = 1024
    o = u2.shape[1]

    def body(q_ref, u2_ref, dinv_ref, b2_ref, z_ref):
        z_ref[...] = (dinv_ref[...] * (q_ref[0] + q_ref[1] + u2_ref[...])
                      + b2_ref[...])

    return pl.pallas_call(
        body,
        grid=(n_pad // bn,),
        in_specs=[
            pl.BlockSpec((NC, bn, o), lambda i: (0, i, 0)),
            pl.BlockSpec((bn, o), lambda i: (i, 0)),
            pl.BlockSpec((bn, 1), lambda i: (i, 0)),
            pl.BlockSpec((1, o), lambda i: (0, 0)),
        ],
        out_specs=pl.BlockSpec((bn, o), lambda i: (i, 0)),
        out_shape=jax.ShapeDtypeStruct((n_pad, o), jnp.float32),
    )(q, u2, dinv, b2r)


def kernel(x, edge_index, W1, b1, W2, b2):
    n, f = x.shape
    e = edge_index.shape[1]
    h = W1.shape[1]
    o = W2.shape[1]

    grain = NW * CH * NBUF
    e_pad = -(-e // grain) * grain
    nch = e_pad // (NW * CH)
    n_pad = -(-(n + 1) // 1024) * 1024

    x_p = jnp.pad(x, ((0, n_pad - n), (0, 0)))
    src = jnp.pad(edge_index[0], (0, e_pad - e)).reshape(NW, nch, CH)
    dst = jnp.pad(edge_index[1], (0, e_pad - e),
                  constant_values=n).reshape(NW, nch, CH)

    zeros_h = jnp.zeros((n_pad, h), jnp.float32)
    zeros_o = jnp.zeros((n_pad, o), jnp.float32)

    deg_parts = _deg_sc(dst.reshape(-1), n_pad, nch)
    u1, dinv = _encode_tc(deg_parts, x_p, W1, n_pad)
    p = _scatter_sc(u1, src, dst, zeros_h, n_pad, nch, h)
    u2 = _mid_tc(p, u1, dinv, b1.reshape(1, -1), W2, n_pad)
    q = _scatter_sc(u2, src, dst, zeros_o, n_pad, nch, o)
    z = _final_tc(q, u2, dinv, b2.reshape(1, -1), n_pad)
    value = _decode_sc(z, src, dst, nch, o)
    return value.reshape(-1)[:e]
